# baseline (speedup 1.0000x reference)
import functools

import jax
import jax.numpy as jnp
from jax import lax
from jax.experimental import pallas as pl
from jax.experimental.pallas import tpu as pltpu
from jax.experimental.pallas import tpu_sc as plsc

D_MODEL = 1024
D_FF = 4096
N_EXPERTS = 8
N_TOKENS = 4096
CAPACITY = int(N_TOKENS / N_EXPERTS * 1.25) + 1
C_PAD = 656
S = N_EXPERTS * C_PAD
ZERO_SLOT = C_PAD - 1
ZERO_BASE = C_PAD - 8

SC_NC = 2
SC_NS = 16
SC_NW = SC_NC * SC_NS
S_PAD = S + SC_NW
_WPAD = 128


_G_BLK = 1024


def _gate_body(x_ref, wg_ref, sidx_ref, cidx_ref, w16_ref,
               carry_ref, tril_ref):
    i = pl.program_id(0)

    @pl.when(i == 0)
    def _():
        carry_ref[...] = jnp.zeros_like(carry_ref)
        r = lax.broadcasted_iota(jnp.int32, (_G_BLK, _G_BLK), 0)
        c = lax.broadcasted_iota(jnp.int32, (_G_BLK, _G_BLK), 1)
        tril_ref[...] = (r >= c).astype(jnp.bfloat16)

    logits = lax.dot_general(
        x_ref[...], wg_ref[...], (((1,), (0,)), ((), ())),
        preferred_element_type=jnp.float32)
    m = jnp.max(logits, axis=-1, keepdims=True)
    e = jnp.exp(logits - m)
    w = jnp.max(e, axis=-1, keepdims=True) / jnp.sum(e, axis=-1, keepdims=True)
    lane = lax.broadcasted_iota(jnp.int32, logits.shape, 1)
    top1 = jnp.min(jnp.where(logits == m, lane, N_EXPERTS), axis=-1,
                   keepdims=True)
    oh = (lane == top1).astype(jnp.float32)
    pos = lax.dot_general(
        tril_ref[...], oh.astype(jnp.bfloat16), (((1,), (0,)), ((), ())),
        preferred_element_type=jnp.float32) + carry_ref[...]
    carry_ref[...] = carry_ref[...] + jnp.sum(oh, axis=0, keepdims=True)
    pos_i = jnp.sum(pos * oh, axis=1, keepdims=True).astype(jnp.int32)
    kept = pos_i <= CAPACITY
    slot = top1 * C_PAD + pos_i - 1
    sidx_ref[...] = jnp.where(kept, slot, S)[:, 0]
    cidx_ref[...] = jnp.where(kept, slot, ZERO_SLOT)[:, 0]
    w16_ref[...] = jnp.broadcast_to(w, (_G_BLK, _WPAD))


def _gating(x_flat, Wg):
    grid = N_TOKENS // _G_BLK
    return pl.pallas_call(
        _gate_body,
        grid=(grid,),
        in_specs=[
            pl.BlockSpec((_G_BLK, D_MODEL), lambda i: (i, 0)),
            pl.BlockSpec((D_MODEL, N_EXPERTS), lambda i: (0, 0)),
        ],
        out_specs=[
            pl.BlockSpec((_G_BLK,), lambda i: (i,)),
            pl.BlockSpec((_G_BLK,), lambda i: (i,)),
            pl.BlockSpec((_G_BLK, _WPAD), lambda i: (i, 0)),
        ],
        out_shape=[
            jax.ShapeDtypeStruct((N_TOKENS,), jnp.int32),
            jax.ShapeDtypeStruct((N_TOKENS,), jnp.int32),
            jax.ShapeDtypeStruct((N_TOKENS, _WPAD), jnp.float32),
        ],
        scratch_shapes=[
            pltpu.VMEM((1, N_EXPERTS), jnp.float32),
            pltpu.VMEM((_G_BLK, _G_BLK), jnp.bfloat16),
        ],
        compiler_params=pltpu.CompilerParams(
            dimension_semantics=("arbitrary",)),
    )(x_flat, Wg)


_D_CHUNK = 32
_D_NCH = N_TOKENS // SC_NW // _D_CHUNK


@functools.lru_cache(maxsize=None)
def _make_dispatch():
    mesh = plsc.VectorSubcoreMesh(core_axis_name="c", subcore_axis_name="s")
    per_w = N_TOKENS // SC_NW

    @functools.partial(
        pl.kernel, mesh=mesh,
        out_type=[
            jax.ShapeDtypeStruct((S_PAD, D_MODEL), jnp.float32),
            jax.ShapeDtypeStruct((S_PAD, _WPAD), jnp.float32),
        ],
        scratch_types=[
            pltpu.VMEM((_D_NCH, _D_CHUNK), jnp.int32),
            pltpu.VMEM((3, _D_CHUNK, D_MODEL), jnp.float32),
            pltpu.VMEM((_D_NCH, _D_CHUNK, _WPAD), jnp.float32),
            pltpu.VMEM((8, _WPAD), jnp.float32),
            pltpu.VMEM((8, D_MODEL), jnp.float32),
            pltpu.SemaphoreType.DMA((3,)),
            pltpu.SemaphoreType.DMA((3,)),
            pltpu.SemaphoreType.DMA,
        ],
    )
    def disp_k(x_hbm, w16_hbm, sidx_hbm, xd_hbm, wd_hbm,
               idx_v, rows_v, w16_v, z_v, zbf_v, gsem, wsem, msem):
        wid = lax.axis_index("s") * SC_NC + lax.axis_index("c")
        base = wid * per_w
        for c in range(min(3, _D_NCH)):
            pltpu.async_copy(x_hbm.at[pl.ds(base + c * _D_CHUNK, _D_CHUNK)],
                             rows_v.at[c % 3], gsem.at[c % 3])
        for c in range(_D_NCH):
            pltpu.async_copy(
                sidx_hbm.at[pl.ds(base + c * _D_CHUNK, _D_CHUNK)],
                idx_v.at[c], msem)
            pltpu.async_copy(
                w16_hbm.at[pl.ds(base + c * _D_CHUNK, _D_CHUNK)],
                w16_v.at[c], msem)

        @pl.when(wid == 0)
        def _():
            for r in range(8):
                def zb(j, _, r=r):
                    z_v[r, pl.ds(j * 16, 16)] = jnp.zeros((16,), jnp.float32)
                    return 0

                lax.fori_loop(0, _WPAD // 16, zb, 0)

                def zbb(j, _, r=r):
                    zbf_v[r, pl.ds(j * 16, 16)] = jnp.zeros((16,),
                                                            jnp.float32)
                    return 0

                lax.fori_loop(0, D_MODEL // 16, zbb, 0)

        for c in range(_D_NCH):
            pltpu.make_async_copy(
                sidx_hbm.at[pl.ds(base + c * _D_CHUNK, _D_CHUNK)],
                idx_v.at[c], msem).wait()
            pltpu.make_async_copy(
                w16_hbm.at[pl.ds(base + c * _D_CHUNK, _D_CHUNK)],
                w16_v.at[c], msem).wait()
        for c in range(_D_NCH):
            for j in range(_D_CHUNK // 16):
                v = idx_v[c, pl.ds(j * 16, 16)]
                idx_v[c, pl.ds(j * 16, 16)] = jnp.where(v >= S, S + wid, v)
        for c in range(_D_NCH):
            pltpu.async_copy(w16_v.at[c], wd_hbm.at[idx_v.at[c]], msem)
        for c in range(_D_NCH):
            b = c % 3
            pltpu.make_async_copy(x_hbm.at[pl.ds(base + c * _D_CHUNK,
                                                 _D_CHUNK)],
                                  rows_v.at[b], gsem.at[b]).wait()
            pltpu.async_copy(rows_v.at[b], xd_hbm.at[idx_v.at[c]], wsem.at[b])
            if c + 3 < _D_NCH:
                pltpu.make_async_copy(rows_v.at[b], xd_hbm.at[idx_v.at[c]],
                                      wsem.at[b]).wait()
                pltpu.async_copy(
                    x_hbm.at[pl.ds(base + (c + 3) * _D_CHUNK, _D_CHUNK)],
                    rows_v.at[b], gsem.at[b])
        for c in range(max(0, _D_NCH - 3), _D_NCH):
            b = c % 3
            pltpu.make_async_copy(rows_v.at[b], xd_hbm.at[idx_v.at[c]],
                                  wsem.at[b]).wait()
        for c in range(_D_NCH):
            pltpu.make_async_copy(w16_v.at[c], wd_hbm.at[idx_v.at[c]],
                                  msem).wait()

        @pl.when(wid == 0)
        def _():
            pltpu.sync_copy(zbf_v, xd_hbm.at[pl.ds(ZERO_BASE, 8)])
            pltpu.sync_copy(z_v, wd_hbm.at[pl.ds(ZERO_BASE, 8)])

    return disp_k


@functools.lru_cache(maxsize=None)
def _make_combine(chunk=32):
    per_w = N_TOKENS // SC_NW
    n_chunks = per_w // chunk
    mesh = plsc.VectorSubcoreMesh(core_axis_name="c", subcore_axis_name="s")

    @functools.partial(
        pl.kernel, mesh=mesh,
        out_type=jax.ShapeDtypeStruct((N_TOKENS, D_MODEL), jnp.float32),
        scratch_types=[
            pltpu.VMEM((n_chunks, chunk), jnp.int32),
            pltpu.VMEM((3, chunk, D_MODEL), jnp.float32),
            pltpu.SemaphoreType.DMA((3,)),
            pltpu.SemaphoreType.DMA((3,)),
        ],
    )
    def gather_k(table_hbm, idx_hbm, out_hbm, idx_v, rows_v, gsem, wsem):
        wid = lax.axis_index("s") * SC_NC + lax.axis_index("c")
        base = wid * per_w
        for c in range(n_chunks):
            pltpu.sync_copy(idx_hbm.at[pl.ds(base + c * chunk, chunk)],
                            idx_v.at[c])
        for c in range(min(3, n_chunks)):
            pltpu.async_copy(table_hbm.at[idx_v.at[c]], rows_v.at[c % 3],
                             gsem.at[c % 3])
        for c in range(n_chunks):
            b = c % 3
            pltpu.make_async_copy(table_hbm.at[idx_v.at[c]], rows_v.at[b],
                                  gsem.at[b]).wait()
            pltpu.async_copy(rows_v.at[b],
                             out_hbm.at[pl.ds(base + c * chunk, chunk)],
                             wsem.at[b])
            if c + 3 < n_chunks:
                pltpu.make_async_copy(rows_v.at[b],
                                      out_hbm.at[pl.ds(base + c * chunk,
                                                       chunk)],
                                      wsem.at[b]).wait()
                pltpu.async_copy(table_hbm.at[idx_v.at[c + 3]], rows_v.at[b],
                                 gsem.at[b])
        for c in range(max(0, n_chunks - 3), n_chunks):
            b = c % 3
            pltpu.make_async_copy(rows_v.at[b],
                                  out_hbm.at[pl.ds(base + c * chunk, chunk)],
                                  wsem.at[b]).wait()

    return gather_k


_F_BLK = 2048
_NF = D_FF // _F_BLK


def _ffn_body(xd_ref, w1_ref, b1_ref, w2_ref, b2_ref, wd_ref, out_ref,
              xbf_ref):
    f = pl.program_id(1)

    @pl.when(f == 0)
    def _():
        xbf_ref[...] = xd_ref[...].astype(jnp.bfloat16)

    h = jnp.dot(xbf_ref[...], w1_ref[0].astype(jnp.bfloat16),
                preferred_element_type=jnp.float32) + b1_ref[0]
    h = 0.5 * h * (1.0 + lax.erf(h * 0.7071067811865476))
    part = jnp.dot(h.astype(jnp.bfloat16), w2_ref[0].astype(jnp.bfloat16),
                   preferred_element_type=jnp.float32)

    @pl.when(f == 0)
    def _():
        out_ref[...] = part

    @pl.when(f > 0)
    def _():
        out_ref[...] = out_ref[...] + part

    @pl.when(f == _NF - 1)
    def _():
        out_ref[...] = (out_ref[...] + b2_ref[0]) * wd_ref[:, 0:1]


def _expert_ffn(xd, W1, b1, W2, b2, wd16):
    return pl.pallas_call(
        _ffn_body,
        grid=(N_EXPERTS, _NF),
        in_specs=[
            pl.BlockSpec((C_PAD, D_MODEL), lambda e, f: (e, 0)),
            pl.BlockSpec((1, D_MODEL, _F_BLK), lambda e, f: (e, 0, f)),
            pl.BlockSpec((1, 1, _F_BLK), lambda e, f: (e, 0, f)),
            pl.BlockSpec((1, _F_BLK, D_MODEL), lambda e, f: (e, f, 0)),
            pl.BlockSpec((1, 1, D_MODEL), lambda e, f: (e, 0, 0)),
            pl.BlockSpec((C_PAD, _WPAD), lambda e, f: (e, 0)),
        ],
        out_specs=pl.BlockSpec((C_PAD, D_MODEL), lambda e, f: (e, 0)),
        out_shape=jax.ShapeDtypeStruct((S, D_MODEL), jnp.float32),
        scratch_shapes=[pltpu.VMEM((C_PAD, D_MODEL), jnp.bfloat16)],
        compiler_params=pltpu.CompilerParams(
            dimension_semantics=("parallel", "arbitrary")),
    )(xd, W1, b1.reshape(N_EXPERTS, 1, D_FF), W2,
      b2.reshape(N_EXPERTS, 1, D_MODEL), wd16)


def kernel(x, Wg, W1, b1, W2, b2):
    B, T, D = x.shape
    x_flat = x.reshape(B * T, D)

    sidx, cidx, w16 = _gating(x_flat, Wg)
    xd, wd16 = _make_dispatch()(x_flat, w16, sidx)
    y = _expert_ffn(xd, W1, b1, W2, b2, wd16)
    out_flat = _make_combine()(y, cidx)
    return out_flat.reshape(B, T, D)


# speedup vs baseline: 1.0101x; 1.0101x over previous
import functools

import jax
import jax.numpy as jnp
from jax import lax
from jax.experimental import pallas as pl
from jax.experimental.pallas import tpu as pltpu
from jax.experimental.pallas import tpu_sc as plsc

D_MODEL = 1024
D_FF = 4096
N_EXPERTS = 8
N_TOKENS = 4096
CAPACITY = int(N_TOKENS / N_EXPERTS * 1.25) + 1
C_PAD = 656
S = N_EXPERTS * C_PAD
ZERO_SLOT = C_PAD - 1
ZERO_BASE = C_PAD - 8

SC_NC = 2
SC_NS = 16
SC_NW = SC_NC * SC_NS
S_PAD = S + SC_NW
_WPAD = 128


_G_BLK = 1024


def _gate_body(x_ref, wg_ref, sidx_ref, cidx_ref, w16_ref,
               carry_ref, tril_ref):
    i = pl.program_id(0)

    @pl.when(i == 0)
    def _():
        carry_ref[...] = jnp.zeros_like(carry_ref)
        r = lax.broadcasted_iota(jnp.int32, (_G_BLK, _G_BLK), 0)
        c = lax.broadcasted_iota(jnp.int32, (_G_BLK, _G_BLK), 1)
        tril_ref[...] = (r >= c).astype(jnp.bfloat16)

    logits = lax.dot_general(
        x_ref[...], wg_ref[...], (((1,), (0,)), ((), ())),
        preferred_element_type=jnp.float32)
    m = jnp.max(logits, axis=-1, keepdims=True)
    e = jnp.exp(logits - m)
    w = jnp.max(e, axis=-1, keepdims=True) / jnp.sum(e, axis=-1, keepdims=True)
    lane = lax.broadcasted_iota(jnp.int32, logits.shape, 1)
    top1 = jnp.min(jnp.where(logits == m, lane, N_EXPERTS), axis=-1,
                   keepdims=True)
    oh = (lane == top1).astype(jnp.float32)
    pos = lax.dot_general(
        tril_ref[...], oh.astype(jnp.bfloat16), (((1,), (0,)), ((), ())),
        preferred_element_type=jnp.float32) + carry_ref[...]
    carry_ref[...] = carry_ref[...] + jnp.sum(oh, axis=0, keepdims=True)
    pos_i = jnp.sum(pos * oh, axis=1, keepdims=True).astype(jnp.int32)
    kept = pos_i <= CAPACITY
    slot = top1 * C_PAD + pos_i - 1
    sidx_ref[...] = jnp.where(kept, slot, S)
    cidx_ref[...] = jnp.where(kept, slot, ZERO_SLOT)
    w16_ref[...] = jnp.broadcast_to(w, (_G_BLK, _WPAD))


def _gating(x_flat, Wg):
    grid = N_TOKENS // _G_BLK
    return pl.pallas_call(
        _gate_body,
        grid=(grid,),
        in_specs=[
            pl.BlockSpec((_G_BLK, D_MODEL), lambda i: (i, 0)),
            pl.BlockSpec((D_MODEL, N_EXPERTS), lambda i: (0, 0)),
        ],
        out_specs=[
            pl.BlockSpec((_G_BLK, 1), lambda i: (i, 0)),
            pl.BlockSpec((_G_BLK, 1), lambda i: (i, 0)),
            pl.BlockSpec((_G_BLK, _WPAD), lambda i: (i, 0)),
        ],
        out_shape=[
            jax.ShapeDtypeStruct((N_TOKENS, 1), jnp.int32),
            jax.ShapeDtypeStruct((N_TOKENS, 1), jnp.int32),
            jax.ShapeDtypeStruct((N_TOKENS, _WPAD), jnp.float32),
        ],
        scratch_shapes=[
            pltpu.VMEM((1, N_EXPERTS), jnp.float32),
            pltpu.VMEM((_G_BLK, _G_BLK), jnp.bfloat16),
        ],
        compiler_params=pltpu.CompilerParams(
            dimension_semantics=("arbitrary",)),
    )(x_flat, Wg)


_D_CHUNK = 32
_D_NCH = N_TOKENS // SC_NW // _D_CHUNK


@functools.lru_cache(maxsize=None)
def _make_dispatch():
    mesh = plsc.VectorSubcoreMesh(core_axis_name="c", subcore_axis_name="s")
    per_w = N_TOKENS // SC_NW

    @functools.partial(
        pl.kernel, mesh=mesh,
        out_type=[
            jax.ShapeDtypeStruct((S_PAD, D_MODEL), jnp.float32),
            jax.ShapeDtypeStruct((S_PAD, _WPAD), jnp.float32),
        ],
        scratch_types=[
            pltpu.VMEM((_D_NCH, _D_CHUNK), jnp.int32),
            pltpu.VMEM((3, _D_CHUNK, D_MODEL), jnp.float32),
            pltpu.VMEM((_D_NCH, _D_CHUNK, _WPAD), jnp.float32),
            pltpu.VMEM((8, _WPAD), jnp.float32),
            pltpu.VMEM((8, D_MODEL), jnp.float32),
            pltpu.SemaphoreType.DMA((3,)),
            pltpu.SemaphoreType.DMA((3,)),
            pltpu.SemaphoreType.DMA,
        ],
    )
    def disp_k(x_hbm, w16_hbm, sidx_hbm, xd_hbm, wd_hbm,
               idx_v, rows_v, w16_v, z_v, zbf_v, gsem, wsem, msem):
        wid = lax.axis_index("s") * SC_NC + lax.axis_index("c")
        base = wid * per_w
        for c in range(min(3, _D_NCH)):
            pltpu.async_copy(x_hbm.at[pl.ds(base + c * _D_CHUNK, _D_CHUNK)],
                             rows_v.at[c % 3], gsem.at[c % 3])
        for c in range(_D_NCH):
            pltpu.async_copy(
                sidx_hbm.at[pl.ds(base + c * _D_CHUNK, _D_CHUNK)],
                idx_v.at[c], msem)
            pltpu.async_copy(
                w16_hbm.at[pl.ds(base + c * _D_CHUNK, _D_CHUNK)],
                w16_v.at[c], msem)

        @pl.when(wid == 0)
        def _():
            for r in range(8):
                def zb(j, _, r=r):
                    z_v[r, pl.ds(j * 16, 16)] = jnp.zeros((16,), jnp.float32)
                    return 0

                lax.fori_loop(0, _WPAD // 16, zb, 0)

                def zbb(j, _, r=r):
                    zbf_v[r, pl.ds(j * 16, 16)] = jnp.zeros((16,),
                                                            jnp.float32)
                    return 0

                lax.fori_loop(0, D_MODEL // 16, zbb, 0)

        for c in range(_D_NCH):
            pltpu.make_async_copy(
                sidx_hbm.at[pl.ds(base + c * _D_CHUNK, _D_CHUNK)],
                idx_v.at[c], msem).wait()
            pltpu.make_async_copy(
                w16_hbm.at[pl.ds(base + c * _D_CHUNK, _D_CHUNK)],
                w16_v.at[c], msem).wait()
        for c in range(_D_NCH):
            for j in range(_D_CHUNK // 16):
                v = idx_v[c, pl.ds(j * 16, 16)]
                idx_v[c, pl.ds(j * 16, 16)] = jnp.where(v >= S, S + wid, v)
        for c in range(_D_NCH):
            pltpu.async_copy(w16_v.at[c], wd_hbm.at[idx_v.at[c]], msem)
        for c in range(_D_NCH):
            b = c % 3
            pltpu.make_async_copy(x_hbm.at[pl.ds(base + c * _D_CHUNK,
                                                 _D_CHUNK)],
                                  rows_v.at[b], gsem.at[b]).wait()
            pltpu.async_copy(rows_v.at[b], xd_hbm.at[idx_v.at[c]], wsem.at[b])
            if c + 3 < _D_NCH:
                pltpu.make_async_copy(rows_v.at[b], xd_hbm.at[idx_v.at[c]],
                                      wsem.at[b]).wait()
                pltpu.async_copy(
                    x_hbm.at[pl.ds(base + (c + 3) * _D_CHUNK, _D_CHUNK)],
                    rows_v.at[b], gsem.at[b])
        for c in range(max(0, _D_NCH - 3), _D_NCH):
            b = c % 3
            pltpu.make_async_copy(rows_v.at[b], xd_hbm.at[idx_v.at[c]],
                                  wsem.at[b]).wait()
        for c in range(_D_NCH):
            pltpu.make_async_copy(w16_v.at[c], wd_hbm.at[idx_v.at[c]],
                                  msem).wait()

        @pl.when(wid == 0)
        def _():
            pltpu.sync_copy(zbf_v, xd_hbm.at[pl.ds(ZERO_BASE, 8)])
            pltpu.sync_copy(z_v, wd_hbm.at[pl.ds(ZERO_BASE, 8)])

    return disp_k


@functools.lru_cache(maxsize=None)
def _make_combine(chunk=32):
    per_w = N_TOKENS // SC_NW
    n_chunks = per_w // chunk
    mesh = plsc.VectorSubcoreMesh(core_axis_name="c", subcore_axis_name="s")

    @functools.partial(
        pl.kernel, mesh=mesh,
        out_type=jax.ShapeDtypeStruct((N_TOKENS, D_MODEL), jnp.float32),
        scratch_types=[
            pltpu.VMEM((n_chunks, chunk), jnp.int32),
            pltpu.VMEM((3, chunk, D_MODEL), jnp.float32),
            pltpu.SemaphoreType.DMA((3,)),
            pltpu.SemaphoreType.DMA((3,)),
        ],
    )
    def gather_k(table_hbm, idx_hbm, out_hbm, idx_v, rows_v, gsem, wsem):
        wid = lax.axis_index("s") * SC_NC + lax.axis_index("c")
        base = wid * per_w
        for c in range(n_chunks):
            pltpu.sync_copy(idx_hbm.at[pl.ds(base + c * chunk, chunk)],
                            idx_v.at[c])
        for c in range(min(3, n_chunks)):
            pltpu.async_copy(table_hbm.at[idx_v.at[c]], rows_v.at[c % 3],
                             gsem.at[c % 3])
        for c in range(n_chunks):
            b = c % 3
            pltpu.make_async_copy(table_hbm.at[idx_v.at[c]], rows_v.at[b],
                                  gsem.at[b]).wait()
            pltpu.async_copy(rows_v.at[b],
                             out_hbm.at[pl.ds(base + c * chunk, chunk)],
                             wsem.at[b])
            if c + 3 < n_chunks:
                pltpu.make_async_copy(rows_v.at[b],
                                      out_hbm.at[pl.ds(base + c * chunk,
                                                       chunk)],
                                      wsem.at[b]).wait()
                pltpu.async_copy(table_hbm.at[idx_v.at[c + 3]], rows_v.at[b],
                                 gsem.at[b])
        for c in range(max(0, n_chunks - 3), n_chunks):
            b = c % 3
            pltpu.make_async_copy(rows_v.at[b],
                                  out_hbm.at[pl.ds(base + c * chunk, chunk)],
                                  wsem.at[b]).wait()

    return gather_k


_F_BLK = 2048
_NF = D_FF // _F_BLK


def _ffn_body(xd_ref, w1_ref, b1_ref, w2_ref, b2_ref, wd_ref, out_ref,
              xbf_ref):
    f = pl.program_id(1)

    @pl.when(f == 0)
    def _():
        xbf_ref[...] = xd_ref[...].astype(jnp.bfloat16)

    h = jnp.dot(xbf_ref[...], w1_ref[0].astype(jnp.bfloat16),
                preferred_element_type=jnp.float32) + b1_ref[0]
    h = 0.5 * h * (1.0 + lax.erf(h * 0.7071067811865476))
    part = jnp.dot(h.astype(jnp.bfloat16), w2_ref[0].astype(jnp.bfloat16),
                   preferred_element_type=jnp.float32)

    @pl.when(f == 0)
    def _():
        out_ref[...] = part

    @pl.when(f > 0)
    def _():
        out_ref[...] = out_ref[...] + part

    @pl.when(f == _NF - 1)
    def _():
        out_ref[...] = (out_ref[...] + b2_ref[0]) * wd_ref[:, 0:1]


def _expert_ffn(xd, W1, b1, W2, b2, wd16):
    return pl.pallas_call(
        _ffn_body,
        grid=(N_EXPERTS, _NF),
        in_specs=[
            pl.BlockSpec((C_PAD, D_MODEL), lambda e, f: (e, 0)),
            pl.BlockSpec((1, D_MODEL, _F_BLK), lambda e, f: (e, 0, f)),
            pl.BlockSpec((1, 1, _F_BLK), lambda e, f: (e, 0, f)),
            pl.BlockSpec((1, _F_BLK, D_MODEL), lambda e, f: (e, f, 0)),
            pl.BlockSpec((1, 1, D_MODEL), lambda e, f: (e, 0, 0)),
            pl.BlockSpec((C_PAD, _WPAD), lambda e, f: (e, 0)),
        ],
        out_specs=pl.BlockSpec((C_PAD, D_MODEL), lambda e, f: (e, 0)),
        out_shape=jax.ShapeDtypeStruct((S, D_MODEL), jnp.float32),
        scratch_shapes=[pltpu.VMEM((C_PAD, D_MODEL), jnp.bfloat16)],
        compiler_params=pltpu.CompilerParams(
            dimension_semantics=("parallel", "arbitrary")),
    )(xd, W1, b1.reshape(N_EXPERTS, 1, D_FF), W2,
      b2.reshape(N_EXPERTS, 1, D_MODEL), wd16)


def kernel(x, Wg, W1, b1, W2, b2):
    B, T, D = x.shape
    x_flat = x.reshape(B * T, D)

    sidx, cidx, w16 = _gating(x_flat, Wg)
    xd, wd16 = _make_dispatch()(x_flat, w16, sidx[:, 0])
    y = _expert_ffn(xd, W1, b1, W2, b2, wd16)
    out_flat = _make_combine()(y, cidx[:, 0])
    return out_flat.reshape(B, T, D)
